# Initial kernel scaffold; baseline (speedup 1.0000x reference)
import jax
import jax.numpy as jnp
from jax.experimental import pallas as pl


def kernel(keys, entries, query, wordemb, qproj_w, qproj_b):
    raise NotImplementedError("write your pallas kernel here")


# speedup vs baseline: 1.3136x; 1.3136x over previous
import functools

import jax
import jax.numpy as jnp
from jax.experimental import pallas as pl
from jax.experimental.pallas import tpu as pltpu

_NC = 2
_MBLK = 400
_NEG = -1e30


def _fused_body(nb, ntok, keys_blk, ents_blk, q_t, w, qb,
                s_out, mx_out, den_out, okt_out, cbt_out,
                vqt_s, mx_s, den_s, okt_s, cbt_s):
    j = pl.program_id(1)
    mblk = keys_blk.shape[0]
    d, b = q_t.shape

    @pl.when(j == 0)
    def _init():
        vqt_s[...] = (jnp.dot(w[...], q_t[...],
                              preferred_element_type=jnp.float32) + qb[...])
        mx_s[...] = jnp.full((1, b), _NEG, jnp.float32)
        den_s[...] = jnp.zeros((1, b), jnp.float32)
        okt_s[...] = jnp.zeros((d, b), jnp.float32)
        cbt_s[...] = jnp.zeros((d, b), jnp.float32)

    ke = keys_blk[...]
    ee = ents_blk[...]

    s_t = jnp.dot(ke, q_t[...], preferred_element_type=jnp.float32)
    s_out[...] = s_t

    l_t = jnp.dot(ee, vqt_s[...], preferred_element_type=jnp.float32)
    l3 = l_t.reshape(mblk, ntok, b)
    mi = jnp.max(l3, axis=1, keepdims=True)
    p3 = jnp.exp(l3 - mi)
    si = jnp.sum(p3, axis=1, keepdims=True)

    m_old = mx_s[...]
    m_new = jnp.maximum(m_old, jnp.max(s_t, axis=0, keepdims=True))
    corr = jnp.exp(m_old - m_new)
    p_t = jnp.exp(s_t - m_new)
    mx_s[...] = m_new
    den_s[...] = den_s[...] * corr + jnp.sum(p_t, axis=0, keepdims=True)

    okt_s[...] = okt_s[...] * corr + jax.lax.dot_general(
        ke, p_t, (((0,), (0,)), ((), ())),
        preferred_element_type=jnp.float32)

    w3 = p3 * (p_t.reshape(mblk, 1, b) / si)
    w_t = w3.reshape(mblk * ntok, b)
    cbt_s[...] = cbt_s[...] * corr + jax.lax.dot_general(
        ee, w_t, (((0,), (0,)), ((), ())),
        preferred_element_type=jnp.float32)

    @pl.when(j == nb - 1)
    def _fin():
        mx_out[...] = mx_s[...].reshape(1, 1, b)
        den_out[...] = den_s[...].reshape(1, 1, b)
        okt_out[...] = okt_s[...].reshape(1, d, b)
        cbt_out[...] = cbt_s[...].reshape(1, d, b)


def kernel(keys, entries, query, wordemb, qproj_w, qproj_b):
    m, ntok = entries.shape
    b, d = query.shape
    mblk = _MBLK
    nb = m // (_NC * mblk)
    assert _NC * nb * mblk == m

    keys_e = wordemb[keys] * (keys != 0)[:, None].astype(wordemb.dtype)
    ents_e = (wordemb[entries.reshape(-1)]
              * (entries.reshape(-1) != 0)[:, None].astype(wordemb.dtype))
    q_t = query.T
    qb = qproj_b[:, None]

    grid = (_NC, nb)
    f32 = jnp.float32
    out_shapes = (
        jax.ShapeDtypeStruct((m, b), f32),
        jax.ShapeDtypeStruct((_NC, 1, b), f32),
        jax.ShapeDtypeStruct((_NC, 1, b), f32),
        jax.ShapeDtypeStruct((_NC, d, b), f32),
        jax.ShapeDtypeStruct((_NC, d, b), f32),
    )
    in_specs = [
        pl.BlockSpec((mblk, d), lambda c, j: (c * nb + j, 0)),
        pl.BlockSpec((mblk * ntok, d), lambda c, j: (c * nb + j, 0)),
        pl.BlockSpec((d, b), lambda c, j: (0, 0)),
        pl.BlockSpec((d, d), lambda c, j: (0, 0)),
        pl.BlockSpec((d, 1), lambda c, j: (0, 0)),
    ]
    out_specs = (
        pl.BlockSpec((mblk, b), lambda c, j: (c * nb + j, 0)),
        pl.BlockSpec((1, 1, b), lambda c, j: (c, 0, 0)),
        pl.BlockSpec((1, 1, b), lambda c, j: (c, 0, 0)),
        pl.BlockSpec((1, d, b), lambda c, j: (c, 0, 0)),
        pl.BlockSpec((1, d, b), lambda c, j: (c, 0, 0)),
    )
    scratch = [
        pltpu.VMEM((d, b), f32),
        pltpu.VMEM((1, b), f32),
        pltpu.VMEM((1, b), f32),
        pltpu.VMEM((d, b), f32),
        pltpu.VMEM((d, b), f32),
    ]
    s_t, mx, den, okt, cbt = pl.pallas_call(
        functools.partial(_fused_body, nb, ntok),
        grid=grid,
        in_specs=in_specs,
        out_specs=out_specs,
        out_shape=out_shapes,
        scratch_shapes=scratch,
        compiler_params=pltpu.CompilerParams(
            dimension_semantics=("parallel", "arbitrary"),
            vmem_limit_bytes=60 * 1024 * 1024,
        ),
    )(keys_e, ents_e, q_t, qproj_w, qb)

    gmax = jnp.max(mx, axis=0)
    wgt = jnp.exp(mx - gmax)
    den_tot = jnp.sum(den * wgt, axis=0)
    okt_tot = jnp.sum(okt * wgt, axis=0)
    cbt_tot = jnp.sum(cbt * wgt, axis=0)
    o_k = (okt_tot / den_tot).T
    combined = (cbt_tot / den_tot).T
    logits_e = s_t.T
    return o_k, logits_e, combined


# speedup vs baseline: 3.0792x; 2.3441x over previous
import functools

import jax
import jax.numpy as jnp
from jax.experimental import pallas as pl
from jax.experimental.pallas import tpu as pltpu

_NC = 2
_MBLK = 80
_NEG = -1e30


def _fused_body(nb, ntok, keys2_ref, ents2_ref, q_t, w, qb, we_hbm,
                s_out, mx_out, den_out, okt_out, cbt_out,
                wes, es, ks, vqt_s, mx_s, den_s, okt_s, cbt_s, dsem):
    j = pl.program_id(1)
    mblk = s_out.shape[0]
    mtok = mblk * ntok
    d, b = q_t.shape
    h = d // 2
    se = es.shape[0] // 2
    sk = ks.shape[0] // 2

    @pl.when(j == 0)
    def _init():
        pltpu.make_async_copy(we_hbm, wes, dsem).start()
        pltpu.make_async_copy(we_hbm, wes, dsem).wait()
        vqt_s[...] = (jnp.dot(w[...], q_t[...],
                              preferred_element_type=jnp.float32) + qb[...])
        mx_s[...] = jnp.full((1, b), _NEG, jnp.float32)
        den_s[...] = jnp.zeros((1, b), jnp.float32)
        okt_s[...] = jnp.zeros((d, b), jnp.float32)
        cbt_s[...] = jnp.zeros((d, b), jnp.float32)

    for t in range(mtok):
        idx = pl.multiple_of(ents2_ref[0, 0, t], 2)
        es[t:t + 2 * se:se, :] = wes[pl.ds(idx, 2), :]
    for t in range(mblk):
        idx = pl.multiple_of(keys2_ref[0, 0, t], 2)
        ks[t:t + 2 * sk:sk, :] = wes[pl.ds(idx, 2), :]

    e0 = es[0:mtok, :]
    e1 = es[se:se + mtok, :]
    k0 = ks[0:mblk, :]
    k1 = ks[sk:sk + mblk, :]
    qt = q_t[...]
    vqt = vqt_s[...]

    s_t = (jnp.dot(k0, qt[0:h], preferred_element_type=jnp.float32)
           + jnp.dot(k1, qt[h:d], preferred_element_type=jnp.float32))
    s_out[...] = s_t

    l_t = (jnp.dot(e0, vqt[0:h], preferred_element_type=jnp.float32)
           + jnp.dot(e1, vqt[h:d], preferred_element_type=jnp.float32))
    l3 = l_t.reshape(mblk, ntok, b)
    mi = jnp.max(l3, axis=1, keepdims=True)
    p3 = jnp.exp(l3 - mi)
    si = jnp.sum(p3, axis=1, keepdims=True)

    m_old = mx_s[...]
    m_new = jnp.maximum(m_old, jnp.max(s_t, axis=0, keepdims=True))
    corr = jnp.exp(m_old - m_new)
    p_t = jnp.exp(s_t - m_new)
    mx_s[...] = m_new
    den_s[...] = den_s[...] * corr + jnp.sum(p_t, axis=0, keepdims=True)

    dg = (((0,), (0,)), ((), ()))
    okt_s[0:h, :] = okt_s[0:h, :] * corr + jax.lax.dot_general(
        k0, p_t, dg, preferred_element_type=jnp.float32)
    okt_s[h:d, :] = okt_s[h:d, :] * corr + jax.lax.dot_general(
        k1, p_t, dg, preferred_element_type=jnp.float32)

    w3 = p3 * (p_t.reshape(mblk, 1, b) / si)
    w_t = w3.reshape(mtok, b)
    cbt_s[0:h, :] = cbt_s[0:h, :] * corr + jax.lax.dot_general(
        e0, w_t, dg, preferred_element_type=jnp.float32)
    cbt_s[h:d, :] = cbt_s[h:d, :] * corr + jax.lax.dot_general(
        e1, w_t, dg, preferred_element_type=jnp.float32)

    @pl.when(j == nb - 1)
    def _fin():
        mx_out[...] = mx_s[...].reshape(1, 1, b)
        den_out[...] = den_s[...].reshape(1, 1, b)
        okt_out[...] = okt_s[...].reshape(1, d, b)
        cbt_out[...] = cbt_s[...].reshape(1, d, b)


def kernel(keys, entries, query, wordemb, qproj_w, qproj_b):
    m, ntok = entries.shape
    b, d = query.shape
    v = wordemb.shape[0]
    mblk = _MBLK
    mtok = mblk * ntok
    nb = m // (_NC * mblk)
    assert _NC * nb * mblk == m
    nbt = _NC * nb
    se = (mtok + 8 + 7) // 8 * 8
    sk = (mblk + 8 + 7) // 8 * 8

    we2 = wordemb.at[0].set(0.0).reshape(2 * v, d // 2)
    keys2 = (keys * 2).astype(jnp.int32).reshape(nbt, 1, mblk)
    ents2 = (entries * 2).astype(jnp.int32).reshape(nbt, 1, mtok)
    q_t = query.T
    qb = qproj_b[:, None]

    grid = (_NC, nb)
    f32 = jnp.float32
    out_shapes = (
        jax.ShapeDtypeStruct((m, b), f32),
        jax.ShapeDtypeStruct((_NC, 1, b), f32),
        jax.ShapeDtypeStruct((_NC, 1, b), f32),
        jax.ShapeDtypeStruct((_NC, d, b), f32),
        jax.ShapeDtypeStruct((_NC, d, b), f32),
    )
    in_specs = [
        pl.BlockSpec((1, 1, mblk), lambda c, j: (c * nb + j, 0, 0),
                     memory_space=pltpu.SMEM),
        pl.BlockSpec((1, 1, mtok), lambda c, j: (c * nb + j, 0, 0),
                     memory_space=pltpu.SMEM),
        pl.BlockSpec((d, b), lambda c, j: (0, 0)),
        pl.BlockSpec((d, d), lambda c, j: (0, 0)),
        pl.BlockSpec((d, 1), lambda c, j: (0, 0)),
        pl.BlockSpec(memory_space=pl.ANY),
    ]
    out_specs = (
        pl.BlockSpec((mblk, b), lambda c, j: (c * nb + j, 0)),
        pl.BlockSpec((1, 1, b), lambda c, j: (c, 0, 0)),
        pl.BlockSpec((1, 1, b), lambda c, j: (c, 0, 0)),
        pl.BlockSpec((1, d, b), lambda c, j: (c, 0, 0)),
        pl.BlockSpec((1, d, b), lambda c, j: (c, 0, 0)),
    )
    scratch = [
        pltpu.VMEM((2 * v, d // 2), f32),
        pltpu.VMEM((2 * se, d // 2), f32),
        pltpu.VMEM((2 * sk, d // 2), f32),
        pltpu.VMEM((d, b), f32),
        pltpu.VMEM((1, b), f32),
        pltpu.VMEM((1, b), f32),
        pltpu.VMEM((d, b), f32),
        pltpu.VMEM((d, b), f32),
        pltpu.SemaphoreType.DMA,
    ]
    s_t, mx, den, okt, cbt = pl.pallas_call(
        functools.partial(_fused_body, nb, ntok),
        grid=grid,
        in_specs=in_specs,
        out_specs=out_specs,
        out_shape=out_shapes,
        scratch_shapes=scratch,
        compiler_params=pltpu.CompilerParams(
            dimension_semantics=("parallel", "arbitrary"),
            vmem_limit_bytes=60 * 1024 * 1024,
        ),
    )(keys2, ents2, q_t, qproj_w, qb, we2)

    gmax = jnp.max(mx, axis=0)
    wgt = jnp.exp(mx - gmax)
    den_tot = jnp.sum(den * wgt, axis=0)
    okt_tot = jnp.sum(okt * wgt, axis=0)
    cbt_tot = jnp.sum(cbt * wgt, axis=0)
    o_k = (okt_tot / den_tot).T
    combined = (cbt_tot / den_tot).T
    logits_e = s_t.T
    return o_k, logits_e, combined


# speedup vs baseline: 3.2606x; 1.0589x over previous
import functools

import jax
import jax.numpy as jnp
from jax.experimental import pallas as pl
from jax.experimental.pallas import tpu as pltpu

_NC = 2
_MBLK = 80
_NEG = -1e30


def _fused_body(nb, ntok, keys2_ref, ents2_ref, q_t, w, qb, we_hbm,
                s_out, mx_out, den_out, okt_out, cbt_out,
                wes, es, ks, vqt_s, mx_s, den_s, okt_s, cbt_s, dsem):
    j = pl.program_id(1)
    mblk = s_out.shape[0]
    mtok = mblk * ntok
    d, b = q_t.shape
    h = d // 2
    se = es.shape[0] // 2
    sk = ks.shape[0] // 2

    @pl.when(j == 0)
    def _init():
        pltpu.make_async_copy(we_hbm, wes, dsem).start()
        pltpu.make_async_copy(we_hbm, wes, dsem).wait()
        wes[0:2, :] = jnp.zeros((2, wes.shape[1]), jnp.float32)
        vqt_s[...] = (jnp.dot(w[...], q_t[...],
                              preferred_element_type=jnp.float32) + qb[...])
        mx_s[...] = jnp.full((1, b), _NEG, jnp.float32)
        den_s[...] = jnp.zeros((1, b), jnp.float32)
        okt_s[...] = jnp.zeros((d, b), jnp.float32)
        cbt_s[...] = jnp.zeros((d, b), jnp.float32)

    u = 16
    for t0 in range(0, mtok, u):
        slabs = [wes[pl.ds(pl.multiple_of(ents2_ref[0, 0, t0 + i], 2), 2), :]
                 for i in range(u)]
        for i in range(u):
            es[t0 + i:t0 + i + 2 * se:se, :] = slabs[i]
    for t0 in range(0, mblk, u):
        slabs = [wes[pl.ds(pl.multiple_of(keys2_ref[0, 0, t0 + i], 2), 2), :]
                 for i in range(u)]
        for i in range(u):
            ks[t0 + i:t0 + i + 2 * sk:sk, :] = slabs[i]

    e0 = es[0:mtok, :]
    e1 = es[se:se + mtok, :]
    k0 = ks[0:mblk, :]
    k1 = ks[sk:sk + mblk, :]
    qt = q_t[...]
    vqt = vqt_s[...]

    s_t = (jnp.dot(k0, qt[0:h], preferred_element_type=jnp.float32)
           + jnp.dot(k1, qt[h:d], preferred_element_type=jnp.float32))
    s_out[...] = s_t

    l_t = (jnp.dot(e0, vqt[0:h], preferred_element_type=jnp.float32)
           + jnp.dot(e1, vqt[h:d], preferred_element_type=jnp.float32))
    l3 = l_t.reshape(mblk, ntok, b)
    mi = jnp.max(l3, axis=1, keepdims=True)
    p3 = jnp.exp(l3 - mi)
    si = jnp.sum(p3, axis=1, keepdims=True)

    m_old = mx_s[...]
    m_new = jnp.maximum(m_old, jnp.max(s_t, axis=0, keepdims=True))
    corr = jnp.exp(m_old - m_new)
    p_t = jnp.exp(s_t - m_new)
    mx_s[...] = m_new
    den_s[...] = den_s[...] * corr + jnp.sum(p_t, axis=0, keepdims=True)

    dg = (((0,), (0,)), ((), ()))
    okt_s[0:h, :] = okt_s[0:h, :] * corr + jax.lax.dot_general(
        k0, p_t, dg, preferred_element_type=jnp.float32)
    okt_s[h:d, :] = okt_s[h:d, :] * corr + jax.lax.dot_general(
        k1, p_t, dg, preferred_element_type=jnp.float32)

    w3 = p3 * (p_t.reshape(mblk, 1, b) / si)
    w_t = w3.reshape(mtok, b)
    cbt_s[0:h, :] = cbt_s[0:h, :] * corr + jax.lax.dot_general(
        e0, w_t, dg, preferred_element_type=jnp.float32)
    cbt_s[h:d, :] = cbt_s[h:d, :] * corr + jax.lax.dot_general(
        e1, w_t, dg, preferred_element_type=jnp.float32)

    @pl.when(j == nb - 1)
    def _fin():
        mx_out[...] = mx_s[...].reshape(1, 1, b)
        den_out[...] = den_s[...].reshape(1, 1, b)
        okt_out[...] = okt_s[...].reshape(1, d, b)
        cbt_out[...] = cbt_s[...].reshape(1, d, b)


def kernel(keys, entries, query, wordemb, qproj_w, qproj_b):
    m, ntok = entries.shape
    b, d = query.shape
    v = wordemb.shape[0]
    mblk = _MBLK
    mtok = mblk * ntok
    nb = m // (_NC * mblk)
    assert _NC * nb * mblk == m
    nbt = _NC * nb
    se = (mtok + 8 + 7) // 8 * 8
    sk = (mblk + 8 + 7) // 8 * 8

    we2 = wordemb.reshape(2 * v, d // 2)
    keys2 = (keys * 2).astype(jnp.int32).reshape(nbt, 1, mblk)
    ents2 = (entries * 2).astype(jnp.int32).reshape(nbt, 1, mtok)
    q_t = query.T
    qb = qproj_b[:, None]

    grid = (_NC, nb)
    f32 = jnp.float32
    out_shapes = (
        jax.ShapeDtypeStruct((m, b), f32),
        jax.ShapeDtypeStruct((_NC, 1, b), f32),
        jax.ShapeDtypeStruct((_NC, 1, b), f32),
        jax.ShapeDtypeStruct((_NC, d, b), f32),
        jax.ShapeDtypeStruct((_NC, d, b), f32),
    )
    in_specs = [
        pl.BlockSpec((1, 1, mblk), lambda c, j: (c * nb + j, 0, 0),
                     memory_space=pltpu.SMEM),
        pl.BlockSpec((1, 1, mtok), lambda c, j: (c * nb + j, 0, 0),
                     memory_space=pltpu.SMEM),
        pl.BlockSpec((d, b), lambda c, j: (0, 0)),
        pl.BlockSpec((d, d), lambda c, j: (0, 0)),
        pl.BlockSpec((d, 1), lambda c, j: (0, 0)),
        pl.BlockSpec(memory_space=pl.ANY),
    ]
    out_specs = (
        pl.BlockSpec((mblk, b), lambda c, j: (c * nb + j, 0)),
        pl.BlockSpec((1, 1, b), lambda c, j: (c, 0, 0)),
        pl.BlockSpec((1, 1, b), lambda c, j: (c, 0, 0)),
        pl.BlockSpec((1, d, b), lambda c, j: (c, 0, 0)),
        pl.BlockSpec((1, d, b), lambda c, j: (c, 0, 0)),
    )
    scratch = [
        pltpu.VMEM((2 * v, d // 2), f32),
        pltpu.VMEM((2 * se, d // 2), f32),
        pltpu.VMEM((2 * sk, d // 2), f32),
        pltpu.VMEM((d, b), f32),
        pltpu.VMEM((1, b), f32),
        pltpu.VMEM((1, b), f32),
        pltpu.VMEM((d, b), f32),
        pltpu.VMEM((d, b), f32),
        pltpu.SemaphoreType.DMA,
    ]
    s_t, mx, den, okt, cbt = pl.pallas_call(
        functools.partial(_fused_body, nb, ntok),
        grid=grid,
        in_specs=in_specs,
        out_specs=out_specs,
        out_shape=out_shapes,
        scratch_shapes=scratch,
        compiler_params=pltpu.CompilerParams(
            dimension_semantics=("parallel", "arbitrary"),
            vmem_limit_bytes=60 * 1024 * 1024,
        ),
    )(keys2, ents2, q_t, qproj_w, qb, we2)

    gmax = jnp.max(mx, axis=0)
    wgt = jnp.exp(mx - gmax)
    den_tot = jnp.sum(den * wgt, axis=0)
    okt_tot = jnp.sum(okt * wgt, axis=0)
    cbt_tot = jnp.sum(cbt * wgt, axis=0)
    o_k = (okt_tot / den_tot).T
    combined = (cbt_tot / den_tot).T
    logits_e = s_t.T
    return o_k, logits_e, combined


# speedup vs baseline: 3.3846x; 1.0380x over previous
import functools

import jax
import jax.numpy as jnp
from jax.experimental import pallas as pl
from jax.experimental.pallas import tpu as pltpu

_NC = 2
_MBLK = 80
_NEG = -1e30
_U = 16


def _gather(keys2_ref, ents2_ref, wes, es, ks):
    se = es.shape[0] // 2
    sk = ks.shape[0] // 2
    mblk = sk - 8
    mtok = se - 8
    for t0 in range(0, mtok, _U):
        slabs = [wes[pl.ds(pl.multiple_of(ents2_ref[0, 0, t0 + i], 2), 2), :]
                 for i in range(_U)]
        for i in range(_U):
            es[t0 + i:t0 + i + 2 * se:se, :] = slabs[i]
    for t0 in range(0, mblk, _U):
        slabs = [wes[pl.ds(pl.multiple_of(keys2_ref[0, 0, t0 + i], 2), 2), :]
                 for i in range(_U)]
        for i in range(_U):
            ks[t0 + i:t0 + i + 2 * sk:sk, :] = slabs[i]


def _compute(ntok, es, ks, q_t, s_out,
             vqt_s, mx_s, den_s, okt_s, cbt_s):
    se = es.shape[0] // 2
    sk = ks.shape[0] // 2
    mblk = sk - 8
    mtok = se - 8
    d, b = q_t.shape
    h = d // 2

    e0 = es[0:mtok, :]
    e1 = es[se:se + mtok, :]
    k0 = ks[0:mblk, :]
    k1 = ks[sk:sk + mblk, :]
    qt = q_t[...]
    vqt = vqt_s[...]

    s_t = (jnp.dot(k0, qt[0:h], preferred_element_type=jnp.float32)
           + jnp.dot(k1, qt[h:d], preferred_element_type=jnp.float32))
    s_out[...] = s_t

    l_t = (jnp.dot(e0, vqt[0:h], preferred_element_type=jnp.float32)
           + jnp.dot(e1, vqt[h:d], preferred_element_type=jnp.float32))
    l3 = l_t.reshape(mblk, ntok, b)
    mi = jnp.max(l3, axis=1, keepdims=True)
    p3 = jnp.exp(l3 - mi)
    si = jnp.sum(p3, axis=1, keepdims=True)

    m_old = mx_s[...]
    m_new = jnp.maximum(m_old, jnp.max(s_t, axis=0, keepdims=True))
    corr = jnp.exp(m_old - m_new)
    p_t = jnp.exp(s_t - m_new)
    mx_s[...] = m_new
    den_s[...] = den_s[...] * corr + jnp.sum(p_t, axis=0, keepdims=True)

    dg = (((0,), (0,)), ((), ()))
    okt_s[0:h, :] = okt_s[0:h, :] * corr + jax.lax.dot_general(
        k0, p_t, dg, preferred_element_type=jnp.float32)
    okt_s[h:d, :] = okt_s[h:d, :] * corr + jax.lax.dot_general(
        k1, p_t, dg, preferred_element_type=jnp.float32)

    w3 = p3 * (p_t.reshape(mblk, 1, b) / si)
    w_t = w3.reshape(mtok, b)
    cbt_s[0:h, :] = cbt_s[0:h, :] * corr + jax.lax.dot_general(
        e0, w_t, dg, preferred_element_type=jnp.float32)
    cbt_s[h:d, :] = cbt_s[h:d, :] * corr + jax.lax.dot_general(
        e1, w_t, dg, preferred_element_type=jnp.float32)


def _fused_body(nb, ntok, keys2_ref, ents2_ref, q_t, w, qb, we_hbm,
                s_out, mx_out, den_out, okt_out, cbt_out,
                wes, es_a, ks_a, es_b, ks_b,
                vqt_s, mx_s, den_s, okt_s, cbt_s, dsem):
    j = pl.program_id(1)
    d, b = q_t.shape
    par0 = (j & 1) == 0

    @pl.when(j == 0)
    def _init():
        pltpu.make_async_copy(we_hbm, wes, dsem).start()
        pltpu.make_async_copy(we_hbm, wes, dsem).wait()
        wes[0:2, :] = jnp.zeros((2, wes.shape[1]), jnp.float32)
        vqt_s[...] = (jnp.dot(w[...], q_t[...],
                              preferred_element_type=jnp.float32) + qb[...])
        mx_s[...] = jnp.full((1, b), _NEG, jnp.float32)
        den_s[...] = jnp.zeros((1, b), jnp.float32)
        okt_s[...] = jnp.zeros((d, b), jnp.float32)
        cbt_s[...] = jnp.zeros((d, b), jnp.float32)

    @pl.when((j < nb) & par0)
    def _g_a():
        _gather(keys2_ref, ents2_ref, wes, es_a, ks_a)

    @pl.when((j < nb) & jnp.logical_not(par0))
    def _g_b():
        _gather(keys2_ref, ents2_ref, wes, es_b, ks_b)

    @pl.when((j > 0) & par0)
    def _c_b():
        _compute(ntok, es_b, ks_b, q_t, s_out,
                 vqt_s, mx_s, den_s, okt_s, cbt_s)

    @pl.when((j > 0) & jnp.logical_not(par0))
    def _c_a():
        _compute(ntok, es_a, ks_a, q_t, s_out,
                 vqt_s, mx_s, den_s, okt_s, cbt_s)

    @pl.when(j == nb)
    def _fin():
        mx_out[...] = mx_s[...].reshape(1, 1, b)
        den_out[...] = den_s[...].reshape(1, 1, b)
        okt_out[...] = okt_s[...].reshape(1, d, b)
        cbt_out[...] = cbt_s[...].reshape(1, d, b)


def kernel(keys, entries, query, wordemb, qproj_w, qproj_b):
    m, ntok = entries.shape
    b, d = query.shape
    v = wordemb.shape[0]
    mblk = _MBLK
    mtok = mblk * ntok
    nb = m // (_NC * mblk)
    assert _NC * nb * mblk == m
    nbt = _NC * nb
    se = mtok + 8
    sk = mblk + 8

    we2 = wordemb.reshape(2 * v, d // 2)
    keys2 = (keys * 2).astype(jnp.int32).reshape(nbt, 1, mblk)
    ents2 = (entries * 2).astype(jnp.int32).reshape(nbt, 1, mtok)
    q_t = query.T
    qb = qproj_b[:, None]

    grid = (_NC, nb + 1)
    f32 = jnp.float32
    out_shapes = (
        jax.ShapeDtypeStruct((m, b), f32),
        jax.ShapeDtypeStruct((_NC, 1, b), f32),
        jax.ShapeDtypeStruct((_NC, 1, b), f32),
        jax.ShapeDtypeStruct((_NC, d, b), f32),
        jax.ShapeDtypeStruct((_NC, d, b), f32),
    )

    def _gmap(c, j):
        return (c * nb + jnp.minimum(j, nb - 1), 0, 0)

    def _cmap(c, j):
        return (c * nb + jnp.maximum(j - 1, 0), 0)

    in_specs = [
        pl.BlockSpec((1, 1, mblk), _gmap, memory_space=pltpu.SMEM),
        pl.BlockSpec((1, 1, mtok), _gmap, memory_space=pltpu.SMEM),
        pl.BlockSpec((d, b), lambda c, j: (0, 0)),
        pl.BlockSpec((d, d), lambda c, j: (0, 0)),
        pl.BlockSpec((d, 1), lambda c, j: (0, 0)),
        pl.BlockSpec(memory_space=pl.ANY),
    ]
    out_specs = (
        pl.BlockSpec((mblk, b), _cmap),
        pl.BlockSpec((1, 1, b), lambda c, j: (c, 0, 0)),
        pl.BlockSpec((1, 1, b), lambda c, j: (c, 0, 0)),
        pl.BlockSpec((1, d, b), lambda c, j: (c, 0, 0)),
        pl.BlockSpec((1, d, b), lambda c, j: (c, 0, 0)),
    )
    scratch = [
        pltpu.VMEM((2 * v, d // 2), f32),
        pltpu.VMEM((2 * se, d // 2), f32),
        pltpu.VMEM((2 * sk, d // 2), f32),
        pltpu.VMEM((2 * se, d // 2), f32),
        pltpu.VMEM((2 * sk, d // 2), f32),
        pltpu.VMEM((d, b), f32),
        pltpu.VMEM((1, b), f32),
        pltpu.VMEM((1, b), f32),
        pltpu.VMEM((d, b), f32),
        pltpu.VMEM((d, b), f32),
        pltpu.SemaphoreType.DMA,
    ]
    s_t, mx, den, okt, cbt = pl.pallas_call(
        functools.partial(_fused_body, nb, ntok),
        grid=grid,
        in_specs=in_specs,
        out_specs=out_specs,
        out_shape=out_shapes,
        scratch_shapes=scratch,
        compiler_params=pltpu.CompilerParams(
            dimension_semantics=("parallel", "arbitrary"),
            vmem_limit_bytes=60 * 1024 * 1024,
        ),
    )(keys2, ents2, q_t, qproj_w, qb, we2)

    gmax = jnp.max(mx, axis=0)
    wgt = jnp.exp(mx - gmax)
    den_tot = jnp.sum(den * wgt, axis=0)
    okt_tot = jnp.sum(okt * wgt, axis=0)
    cbt_tot = jnp.sum(cbt * wgt, axis=0)
    o_k = (okt_tot / den_tot).T
    combined = (cbt_tot / den_tot).T
    logits_e = s_t.T
    return o_k, logits_e, combined


# speedup vs baseline: 3.6390x; 1.0752x over previous
import functools

import jax
import jax.numpy as jnp
from jax.experimental import pallas as pl
from jax.experimental.pallas import tpu as pltpu

_NC = 2
_MBLK = 80
_NEG = -1e30
_U = 16


def _gather(keys2_ref, koff, ents2_ref, eoff, wes, es, ks):
    se = es.shape[0] // 2
    sk = ks.shape[0] // 2
    mblk = sk - 8
    mtok = se - 8
    for t0 in range(0, mtok, _U):
        slabs = [wes[pl.ds(pl.multiple_of(ents2_ref[0, 0, eoff + t0 + i], 2),
                           2), :] for i in range(_U)]
        for i in range(_U):
            es[t0 + i:t0 + i + 2 * se:se, :] = slabs[i]
    for t0 in range(0, mblk, _U):
        slabs = [wes[pl.ds(pl.multiple_of(keys2_ref[0, 0, koff + t0 + i], 2),
                           2), :] for i in range(_U)]
        for i in range(_U):
            ks[t0 + i:t0 + i + 2 * sk:sk, :] = slabs[i]


def _compute(ntok, es, ks, q_t, s_out,
             vqt_s, mx_s, den_s, okt_s, cbt_s):
    se = es.shape[0] // 2
    sk = ks.shape[0] // 2
    mblk = sk - 8
    mtok = se - 8
    d, b = q_t.shape
    h = d // 2

    e0 = es[0:mtok, :]
    e1 = es[se:se + mtok, :]
    k0 = ks[0:mblk, :]
    k1 = ks[sk:sk + mblk, :]
    qt = q_t[...]
    vqt = vqt_s[...]

    s_t = (jnp.dot(k0, qt[0:h], preferred_element_type=jnp.float32)
           + jnp.dot(k1, qt[h:d], preferred_element_type=jnp.float32))
    s_out[...] = s_t

    l_t = (jnp.dot(e0, vqt[0:h], preferred_element_type=jnp.float32)
           + jnp.dot(e1, vqt[h:d], preferred_element_type=jnp.float32))
    l3 = l_t.reshape(mblk, ntok, b)
    mi = jnp.max(l3, axis=1, keepdims=True)
    p3 = jnp.exp(l3 - mi)
    si = jnp.sum(p3, axis=1, keepdims=True)

    m_old = mx_s[...]
    m_new = jnp.maximum(m_old, jnp.max(s_t, axis=0, keepdims=True))
    corr = jnp.exp(m_old - m_new)
    p_t = jnp.exp(s_t - m_new)
    mx_s[...] = m_new
    den_s[...] = den_s[...] * corr + jnp.sum(p_t, axis=0, keepdims=True)

    dg = (((0,), (0,)), ((), ()))
    okt_s[0:h, :] = okt_s[0:h, :] * corr + jax.lax.dot_general(
        k0, p_t, dg, preferred_element_type=jnp.float32)
    okt_s[h:d, :] = okt_s[h:d, :] * corr + jax.lax.dot_general(
        k1, p_t, dg, preferred_element_type=jnp.float32)

    w3 = p3 * (p_t.reshape(mblk, 1, b) / si)
    w_t = w3.reshape(mtok, b)
    cbt_s[0:h, :] = cbt_s[0:h, :] * corr + jax.lax.dot_general(
        e0, w_t, dg, preferred_element_type=jnp.float32)
    cbt_s[h:d, :] = cbt_s[h:d, :] * corr + jax.lax.dot_general(
        e1, w_t, dg, preferred_element_type=jnp.float32)


def _fused_body(nb2, ntok, keys2c, ents2c, keys2n, ents2n, q_t, w, qb,
                we_hbm, s_oute, s_outo, mx_out, den_out, okt_out, cbt_out,
                wes, es_a, ks_a, es_b, ks_b,
                vqt_s, mx_s, den_s, okt_s, cbt_s, dsem):
    g = pl.program_id(1)
    d, b = q_t.shape
    mblk = ks_a.shape[0] // 2 - 8
    mtok = es_a.shape[0] // 2 - 8

    @pl.when(g == 0)
    def _init():
        pltpu.make_async_copy(we_hbm, wes, dsem).start()
        pltpu.make_async_copy(we_hbm, wes, dsem).wait()
        wes[0:2, :] = jnp.zeros((2, wes.shape[1]), jnp.float32)
        vqt_s[...] = (jnp.dot(w[...], q_t[...],
                              preferred_element_type=jnp.float32) + qb[...])
        mx_s[...] = jnp.full((1, b), _NEG, jnp.float32)
        den_s[...] = jnp.zeros((1, b), jnp.float32)
        okt_s[...] = jnp.zeros((d, b), jnp.float32)
        cbt_s[...] = jnp.zeros((d, b), jnp.float32)
        _gather(keys2c, 0, ents2c, 0, wes, es_a, ks_a)

    _gather(keys2c, mblk, ents2c, mtok, wes, es_b, ks_b)
    _compute(ntok, es_a, ks_a, q_t, s_oute,
             vqt_s, mx_s, den_s, okt_s, cbt_s)
    _gather(keys2n, 0, ents2n, 0, wes, es_a, ks_a)
    _compute(ntok, es_b, ks_b, q_t, s_outo,
             vqt_s, mx_s, den_s, okt_s, cbt_s)

    @pl.when(g == nb2 - 1)
    def _fin():
        mx_out[...] = mx_s[...].reshape(1, 1, b)
        den_out[...] = den_s[...].reshape(1, 1, b)
        okt_out[...] = okt_s[...].reshape(1, d, b)
        cbt_out[...] = cbt_s[...].reshape(1, d, b)


def kernel(keys, entries, query, wordemb, qproj_w, qproj_b):
    m, ntok = entries.shape
    b, d = query.shape
    v = wordemb.shape[0]
    mblk = _MBLK
    mtok = mblk * ntok
    nb = m // (_NC * mblk)
    nb2 = nb // 2
    assert _NC * nb2 * 2 * mblk == m
    se = mtok + 8
    sk = mblk + 8

    we2 = wordemb.reshape(2 * v, d // 2)
    keys2 = (keys * 2).astype(jnp.int32).reshape(_NC * nb2, 1, 2 * mblk)
    ents2 = (entries * 2).astype(jnp.int32).reshape(_NC * nb2, 1, 2 * mtok)
    q_t = query.T
    qb = qproj_b[:, None]

    grid = (_NC, nb2)
    f32 = jnp.float32
    out_shapes = (
        jax.ShapeDtypeStruct((_NC * nb2 * mblk, b), f32),
        jax.ShapeDtypeStruct((_NC * nb2 * mblk, b), f32),
        jax.ShapeDtypeStruct((_NC, 1, b), f32),
        jax.ShapeDtypeStruct((_NC, 1, b), f32),
        jax.ShapeDtypeStruct((_NC, d, b), f32),
        jax.ShapeDtypeStruct((_NC, d, b), f32),
    )

    def _cur(c, g):
        return (c * nb2 + g, 0, 0)

    def _nxt(c, g):
        return (c * nb2 + jnp.minimum(g + 1, nb2 - 1), 0, 0)

    in_specs = [
        pl.BlockSpec((1, 1, 2 * mblk), _cur, memory_space=pltpu.SMEM),
        pl.BlockSpec((1, 1, 2 * mtok), _cur, memory_space=pltpu.SMEM),
        pl.BlockSpec((1, 1, 2 * mblk), _nxt, memory_space=pltpu.SMEM),
        pl.BlockSpec((1, 1, 2 * mtok), _nxt, memory_space=pltpu.SMEM),
        pl.BlockSpec((d, b), lambda c, g: (0, 0)),
        pl.BlockSpec((d, d), lambda c, g: (0, 0)),
        pl.BlockSpec((d, 1), lambda c, g: (0, 0)),
        pl.BlockSpec(memory_space=pl.ANY),
    ]
    out_specs = (
        pl.BlockSpec((mblk, b), lambda c, g: (c * nb2 + g, 0)),
        pl.BlockSpec((mblk, b), lambda c, g: (c * nb2 + g, 0)),
        pl.BlockSpec((1, 1, b), lambda c, g: (c, 0, 0)),
        pl.BlockSpec((1, 1, b), lambda c, g: (c, 0, 0)),
        pl.BlockSpec((1, d, b), lambda c, g: (c, 0, 0)),
        pl.BlockSpec((1, d, b), lambda c, g: (c, 0, 0)),
    )
    scratch = [
        pltpu.VMEM((2 * v, d // 2), f32),
        pltpu.VMEM((2 * se, d // 2), f32),
        pltpu.VMEM((2 * sk, d // 2), f32),
        pltpu.VMEM((2 * se, d // 2), f32),
        pltpu.VMEM((2 * sk, d // 2), f32),
        pltpu.VMEM((d, b), f32),
        pltpu.VMEM((1, b), f32),
        pltpu.VMEM((1, b), f32),
        pltpu.VMEM((d, b), f32),
        pltpu.VMEM((d, b), f32),
        pltpu.SemaphoreType.DMA,
    ]
    s_e, s_o, mx, den, okt, cbt = pl.pallas_call(
        functools.partial(_fused_body, nb2, ntok),
        grid=grid,
        in_specs=in_specs,
        out_specs=out_specs,
        out_shape=out_shapes,
        scratch_shapes=scratch,
        compiler_params=pltpu.CompilerParams(
            dimension_semantics=("parallel", "arbitrary"),
            vmem_limit_bytes=60 * 1024 * 1024,
        ),
    )(keys2, ents2, keys2, ents2, q_t, qproj_w, qb, we2)

    se_r = s_e.reshape(_NC, nb2, 1, mblk, b)
    so_r = s_o.reshape(_NC, nb2, 1, mblk, b)
    s_t = jnp.concatenate([se_r, so_r], axis=2).reshape(m, b)

    gmax = jnp.max(mx, axis=0)
    wgt = jnp.exp(mx - gmax)
    den_tot = jnp.sum(den * wgt, axis=0)
    okt_tot = jnp.sum(okt * wgt, axis=0)
    cbt_tot = jnp.sum(cbt * wgt, axis=0)
    o_k = (okt_tot / den_tot).T
    combined = (cbt_tot / den_tot).T
    logits_e = s_t.T
    return o_k, logits_e, combined


# speedup vs baseline: 3.9387x; 1.0824x over previous
import functools

import jax
import jax.numpy as jnp
from jax.experimental import pallas as pl
from jax.experimental.pallas import tpu as pltpu

_NC = 2
_MBLK = 80
_NEG = -1e30
_U = 16


def _gather(keys2_ref, koff, ents2_ref, eoff, wes, es, ks):
    h = wes.shape[1]
    mblk = ks.shape[0]
    mtok = es.shape[0]
    for t0 in range(0, mtok, _U):
        slabs = [wes[pl.ds(pl.multiple_of(ents2_ref[0, 0, eoff + t0 + i], 2),
                           2), :] for i in range(_U)]
        for i in range(_U):
            t = t0 + i
            es[t:t + 1, 0:h] = slabs[i][0:1, :]
            es[t:t + 1, h:2 * h] = slabs[i][1:2, :]
    for t0 in range(0, mblk, _U):
        slabs = [wes[pl.ds(pl.multiple_of(keys2_ref[0, 0, koff + t0 + i], 2),
                           2), :] for i in range(_U)]
        for i in range(_U):
            t = t0 + i
            ks[t:t + 1, 0:h] = slabs[i][0:1, :]
            ks[t:t + 1, h:2 * h] = slabs[i][1:2, :]


def _compute(ntok, es, ks, q_t, s_out,
             vqt_s, mx_s, den_s, okt_s, cbt_s):
    mblk = ks.shape[0]
    mtok = es.shape[0]
    d, b = q_t.shape

    ee = es[...]
    ke = ks[...]
    qt = q_t[...]
    vqt = vqt_s[...]

    s_t = jnp.dot(ke, qt, preferred_element_type=jnp.float32)
    s_out[...] = s_t

    l_t = jnp.dot(ee, vqt, preferred_element_type=jnp.float32)
    l3 = l_t.reshape(mblk, ntok, b)
    mi = jnp.max(l3, axis=1, keepdims=True)
    p3 = jnp.exp(l3 - mi)
    si = jnp.sum(p3, axis=1, keepdims=True)

    m_old = mx_s[...]
    m_new = jnp.maximum(m_old, jnp.max(s_t, axis=0, keepdims=True))
    corr = jnp.exp(m_old - m_new)
    p_t = jnp.exp(s_t - m_new)
    mx_s[...] = m_new
    den_s[...] = den_s[...] * corr + jnp.sum(p_t, axis=0, keepdims=True)

    dg = (((0,), (0,)), ((), ()))
    okt_s[...] = okt_s[...] * corr + jax.lax.dot_general(
        ke, p_t, dg, preferred_element_type=jnp.float32)

    w3 = p3 * (p_t.reshape(mblk, 1, b) / si)
    w_t = w3.reshape(mtok, b)
    cbt_s[...] = cbt_s[...] * corr + jax.lax.dot_general(
        ee, w_t, dg, preferred_element_type=jnp.float32)


def _fused_body(nb2, ntok, keys2c, ents2c, keys2n, ents2n, q_t, w, qb,
                we_hbm, s_oute, s_outo, mx_out, den_out, okt_out, cbt_out,
                wes, es_a, ks_a, es_b, ks_b,
                vqt_s, mx_s, den_s, okt_s, cbt_s, dsem):
    g = pl.program_id(1)
    d, b = q_t.shape
    mblk = ks_a.shape[0]
    mtok = es_a.shape[0]

    @pl.when(g == 0)
    def _init():
        pltpu.make_async_copy(we_hbm, wes, dsem).start()
        pltpu.make_async_copy(we_hbm, wes, dsem).wait()
        wes[0:2, :] = jnp.zeros((2, wes.shape[1]), jnp.float32)
        vqt_s[...] = (jnp.dot(w[...], q_t[...],
                              preferred_element_type=jnp.float32) + qb[...])
        mx_s[...] = jnp.full((1, b), _NEG, jnp.float32)
        den_s[...] = jnp.zeros((1, b), jnp.float32)
        okt_s[...] = jnp.zeros((d, b), jnp.float32)
        cbt_s[...] = jnp.zeros((d, b), jnp.float32)
        _gather(keys2c, 0, ents2c, 0, wes, es_a, ks_a)

    _gather(keys2c, mblk, ents2c, mtok, wes, es_b, ks_b)
    _compute(ntok, es_a, ks_a, q_t, s_oute,
             vqt_s, mx_s, den_s, okt_s, cbt_s)
    _gather(keys2n, 0, ents2n, 0, wes, es_a, ks_a)
    _compute(ntok, es_b, ks_b, q_t, s_outo,
             vqt_s, mx_s, den_s, okt_s, cbt_s)

    @pl.when(g == nb2 - 1)
    def _fin():
        mx_out[...] = mx_s[...].reshape(1, 1, b)
        den_out[...] = den_s[...].reshape(1, 1, b)
        okt_out[...] = okt_s[...].reshape(1, d, b)
        cbt_out[...] = cbt_s[...].reshape(1, d, b)


def kernel(keys, entries, query, wordemb, qproj_w, qproj_b):
    m, ntok = entries.shape
    b, d = query.shape
    v = wordemb.shape[0]
    mblk = _MBLK
    mtok = mblk * ntok
    nb = m // (_NC * mblk)
    nb2 = nb // 2
    assert _NC * nb2 * 2 * mblk == m

    we2 = wordemb.reshape(2 * v, d // 2)
    keys2 = (keys * 2).astype(jnp.int32).reshape(_NC * nb2, 1, 2 * mblk)
    ents2 = (entries * 2).astype(jnp.int32).reshape(_NC * nb2, 1, 2 * mtok)
    q_t = query.T
    qb = qproj_b[:, None]

    grid = (_NC, nb2)
    f32 = jnp.float32
    out_shapes = (
        jax.ShapeDtypeStruct((_NC * nb2 * mblk, b), f32),
        jax.ShapeDtypeStruct((_NC * nb2 * mblk, b), f32),
        jax.ShapeDtypeStruct((_NC, 1, b), f32),
        jax.ShapeDtypeStruct((_NC, 1, b), f32),
        jax.ShapeDtypeStruct((_NC, d, b), f32),
        jax.ShapeDtypeStruct((_NC, d, b), f32),
    )

    def _cur(c, g):
        return (c * nb2 + g, 0, 0)

    def _nxt(c, g):
        return (c * nb2 + jnp.minimum(g + 1, nb2 - 1), 0, 0)

    in_specs = [
        pl.BlockSpec((1, 1, 2 * mblk), _cur, memory_space=pltpu.SMEM),
        pl.BlockSpec((1, 1, 2 * mtok), _cur, memory_space=pltpu.SMEM),
        pl.BlockSpec((1, 1, 2 * mblk), _nxt, memory_space=pltpu.SMEM),
        pl.BlockSpec((1, 1, 2 * mtok), _nxt, memory_space=pltpu.SMEM),
        pl.BlockSpec((d, b), lambda c, g: (0, 0)),
        pl.BlockSpec((d, d), lambda c, g: (0, 0)),
        pl.BlockSpec((d, 1), lambda c, g: (0, 0)),
        pl.BlockSpec(memory_space=pl.ANY),
    ]
    out_specs = (
        pl.BlockSpec((mblk, b), lambda c, g: (c * nb2 + g, 0)),
        pl.BlockSpec((mblk, b), lambda c, g: (c * nb2 + g, 0)),
        pl.BlockSpec((1, 1, b), lambda c, g: (c, 0, 0)),
        pl.BlockSpec((1, 1, b), lambda c, g: (c, 0, 0)),
        pl.BlockSpec((1, d, b), lambda c, g: (c, 0, 0)),
        pl.BlockSpec((1, d, b), lambda c, g: (c, 0, 0)),
    )
    scratch = [
        pltpu.VMEM((2 * v, d // 2), f32),
        pltpu.VMEM((mtok, d), f32),
        pltpu.VMEM((mblk, d), f32),
        pltpu.VMEM((mtok, d), f32),
        pltpu.VMEM((mblk, d), f32),
        pltpu.VMEM((d, b), f32),
        pltpu.VMEM((1, b), f32),
        pltpu.VMEM((1, b), f32),
        pltpu.VMEM((d, b), f32),
        pltpu.VMEM((d, b), f32),
        pltpu.SemaphoreType.DMA,
    ]
    s_e, s_o, mx, den, okt, cbt = pl.pallas_call(
        functools.partial(_fused_body, nb2, ntok),
        grid=grid,
        in_specs=in_specs,
        out_specs=out_specs,
        out_shape=out_shapes,
        scratch_shapes=scratch,
        compiler_params=pltpu.CompilerParams(
            dimension_semantics=("parallel", "arbitrary"),
            vmem_limit_bytes=60 * 1024 * 1024,
        ),
    )(keys2, ents2, keys2, ents2, q_t, qproj_w, qb, we2)

    se_r = s_e.reshape(_NC, nb2, 1, mblk, b)
    so_r = s_o.reshape(_NC, nb2, 1, mblk, b)
    s_t = jnp.concatenate([se_r, so_r], axis=2).reshape(m, b)

    gmax = jnp.max(mx, axis=0)
    wgt = jnp.exp(mx - gmax)
    den_tot = jnp.sum(den * wgt, axis=0)
    okt_tot = jnp.sum(okt * wgt, axis=0)
    cbt_tot = jnp.sum(cbt * wgt, axis=0)
    o_k = (okt_tot / den_tot).T
    combined = (cbt_tot / den_tot).T
    logits_e = s_t.T
    return o_k, logits_e, combined


# speedup vs baseline: 4.1312x; 1.0489x over previous
import functools

import jax
import jax.numpy as jnp
from jax.experimental import pallas as pl
from jax.experimental.pallas import tpu as pltpu

_NC = 1
_MBLK = 80
_NEG = -1e30
_U = 16


def _gather(keys2_ref, koff, ents2_ref, eoff, wes, es, ks):
    h = wes.shape[1]
    mblk = ks.shape[0]
    mtok = es.shape[0]
    for t0 in range(0, mtok, _U):
        slabs = [wes[pl.ds(pl.multiple_of(ents2_ref[0, 0, eoff + t0 + i], 2),
                           2), :] for i in range(_U)]
        for i in range(_U):
            t = t0 + i
            es[t:t + 1, 0:h] = slabs[i][0:1, :]
            es[t:t + 1, h:2 * h] = slabs[i][1:2, :]
    for t0 in range(0, mblk, _U):
        slabs = [wes[pl.ds(pl.multiple_of(keys2_ref[0, 0, koff + t0 + i], 2),
                           2), :] for i in range(_U)]
        for i in range(_U):
            t = t0 + i
            ks[t:t + 1, 0:h] = slabs[i][0:1, :]
            ks[t:t + 1, h:2 * h] = slabs[i][1:2, :]


def _compute(ntok, es, ks, q_t, s_out,
             vqt_s, mx_s, den_s, okt_s, cbt_s):
    mblk = ks.shape[0]
    mtok = es.shape[0]
    d, b = q_t.shape

    ee = es[...]
    ke = ks[...]
    qt = q_t[...]
    vqt = vqt_s[...]

    s_t = jnp.dot(ke, qt, preferred_element_type=jnp.float32)
    s_out[...] = s_t

    l_t = jnp.dot(ee, vqt, preferred_element_type=jnp.float32)
    l3 = l_t.reshape(mblk, ntok, b)
    mi = jnp.max(l3, axis=1, keepdims=True)
    p3 = jnp.exp(l3 - mi)
    si = jnp.sum(p3, axis=1, keepdims=True)

    m_old = mx_s[...]
    m_new = jnp.maximum(m_old, jnp.max(s_t, axis=0, keepdims=True))
    corr = jnp.exp(m_old - m_new)
    p_t = jnp.exp(s_t - m_new)
    mx_s[...] = m_new
    den_s[...] = den_s[...] * corr + jnp.sum(p_t, axis=0, keepdims=True)

    dg = (((0,), (0,)), ((), ()))
    okt_s[...] = okt_s[...] * corr + jax.lax.dot_general(
        ke, p_t, dg, preferred_element_type=jnp.float32)

    w3 = p3 * (p_t.reshape(mblk, 1, b) / si)
    w_t = w3.reshape(mtok, b)
    cbt_s[...] = cbt_s[...] * corr + jax.lax.dot_general(
        ee, w_t, dg, preferred_element_type=jnp.float32)


def _fused_body(nb2, ntok, keys2c, ents2c, keys2n, ents2n, q_t, w, qb,
                we_hbm, s_oute, s_outo, mx_out, den_out, okt_out, cbt_out,
                wes, es_a, ks_a, es_b, ks_b,
                vqt_s, mx_s, den_s, okt_s, cbt_s, dsem):
    g = pl.program_id(1)
    d, b = q_t.shape
    mblk = ks_a.shape[0]
    mtok = es_a.shape[0]

    @pl.when(g == 0)
    def _init():
        pltpu.make_async_copy(we_hbm, wes, dsem).start()
        pltpu.make_async_copy(we_hbm, wes, dsem).wait()
        wes[0:2, :] = jnp.zeros((2, wes.shape[1]), jnp.float32)
        vqt_s[...] = (jnp.dot(w[...], q_t[...],
                              preferred_element_type=jnp.float32) + qb[...])
        mx_s[...] = jnp.full((1, b), _NEG, jnp.float32)
        den_s[...] = jnp.zeros((1, b), jnp.float32)
        okt_s[...] = jnp.zeros((d, b), jnp.float32)
        cbt_s[...] = jnp.zeros((d, b), jnp.float32)
        _gather(keys2c, 0, ents2c, 0, wes, es_a, ks_a)

    _gather(keys2c, mblk, ents2c, mtok, wes, es_b, ks_b)
    _compute(ntok, es_a, ks_a, q_t, s_oute,
             vqt_s, mx_s, den_s, okt_s, cbt_s)
    _gather(keys2n, 0, ents2n, 0, wes, es_a, ks_a)
    _compute(ntok, es_b, ks_b, q_t, s_outo,
             vqt_s, mx_s, den_s, okt_s, cbt_s)

    @pl.when(g == nb2 - 1)
    def _fin():
        mx_out[...] = mx_s[...].reshape(1, 1, b)
        den_out[...] = den_s[...].reshape(1, 1, b)
        okt_out[...] = okt_s[...].reshape(1, d, b)
        cbt_out[...] = cbt_s[...].reshape(1, d, b)


def kernel(keys, entries, query, wordemb, qproj_w, qproj_b):
    m, ntok = entries.shape
    b, d = query.shape
    v = wordemb.shape[0]
    mblk = _MBLK
    mtok = mblk * ntok
    nb = m // (_NC * mblk)
    nb2 = nb // 2
    assert _NC * nb2 * 2 * mblk == m

    we2 = wordemb.reshape(2 * v, d // 2)
    keys2 = (keys * 2).astype(jnp.int32).reshape(_NC * nb2, 1, 2 * mblk)
    ents2 = (entries * 2).astype(jnp.int32).reshape(_NC * nb2, 1, 2 * mtok)
    q_t = query.T
    qb = qproj_b[:, None]

    grid = (_NC, nb2)
    f32 = jnp.float32
    out_shapes = (
        jax.ShapeDtypeStruct((_NC * nb2 * mblk, b), f32),
        jax.ShapeDtypeStruct((_NC * nb2 * mblk, b), f32),
        jax.ShapeDtypeStruct((_NC, 1, b), f32),
        jax.ShapeDtypeStruct((_NC, 1, b), f32),
        jax.ShapeDtypeStruct((_NC, d, b), f32),
        jax.ShapeDtypeStruct((_NC, d, b), f32),
    )

    def _cur(c, g):
        return (c * nb2 + g, 0, 0)

    def _nxt(c, g):
        return (c * nb2 + jnp.minimum(g + 1, nb2 - 1), 0, 0)

    in_specs = [
        pl.BlockSpec((1, 1, 2 * mblk), _cur, memory_space=pltpu.SMEM),
        pl.BlockSpec((1, 1, 2 * mtok), _cur, memory_space=pltpu.SMEM),
        pl.BlockSpec((1, 1, 2 * mblk), _nxt, memory_space=pltpu.SMEM),
        pl.BlockSpec((1, 1, 2 * mtok), _nxt, memory_space=pltpu.SMEM),
        pl.BlockSpec((d, b), lambda c, g: (0, 0)),
        pl.BlockSpec((d, d), lambda c, g: (0, 0)),
        pl.BlockSpec((d, 1), lambda c, g: (0, 0)),
        pl.BlockSpec(memory_space=pl.ANY),
    ]
    out_specs = (
        pl.BlockSpec((mblk, b), lambda c, g: (c * nb2 + g, 0)),
        pl.BlockSpec((mblk, b), lambda c, g: (c * nb2 + g, 0)),
        pl.BlockSpec((1, 1, b), lambda c, g: (c, 0, 0)),
        pl.BlockSpec((1, 1, b), lambda c, g: (c, 0, 0)),
        pl.BlockSpec((1, d, b), lambda c, g: (c, 0, 0)),
        pl.BlockSpec((1, d, b), lambda c, g: (c, 0, 0)),
    )
    scratch = [
        pltpu.VMEM((2 * v, d // 2), f32),
        pltpu.VMEM((mtok, d), f32),
        pltpu.VMEM((mblk, d), f32),
        pltpu.VMEM((mtok, d), f32),
        pltpu.VMEM((mblk, d), f32),
        pltpu.VMEM((d, b), f32),
        pltpu.VMEM((1, b), f32),
        pltpu.VMEM((1, b), f32),
        pltpu.VMEM((d, b), f32),
        pltpu.VMEM((d, b), f32),
        pltpu.SemaphoreType.DMA,
    ]
    s_e, s_o, mx, den, okt, cbt = pl.pallas_call(
        functools.partial(_fused_body, nb2, ntok),
        grid=grid,
        in_specs=in_specs,
        out_specs=out_specs,
        out_shape=out_shapes,
        scratch_shapes=scratch,
        compiler_params=pltpu.CompilerParams(
            dimension_semantics=("parallel", "arbitrary"),
            vmem_limit_bytes=60 * 1024 * 1024,
        ),
    )(keys2, ents2, keys2, ents2, q_t, qproj_w, qb, we2)

    se_r = s_e.reshape(_NC, nb2, 1, mblk, b)
    so_r = s_o.reshape(_NC, nb2, 1, mblk, b)
    s_t = jnp.concatenate([se_r, so_r], axis=2).reshape(m, b)

    gmax = jnp.max(mx, axis=0)
    wgt = jnp.exp(mx - gmax)
    den_tot = jnp.sum(den * wgt, axis=0)
    okt_tot = jnp.sum(okt * wgt, axis=0)
    cbt_tot = jnp.sum(cbt * wgt, axis=0)
    o_k = (okt_tot / den_tot).T
    combined = (cbt_tot / den_tot).T
    logits_e = s_t.T
    return o_k, logits_e, combined
